# Initial kernel scaffold; baseline (speedup 1.0000x reference)
import jax
import jax.numpy as jnp
from jax.experimental import pallas as pl


def kernel(z_e, codebook):
    raise NotImplementedError("write your pallas kernel here")


# speedup vs baseline: 1.1049x; 1.1049x over previous
import jax
import jax.numpy as jnp
from jax import lax
from jax.experimental import pallas as pl
from jax.experimental.pallas import tpu as pltpu

BETA = 0.25
KT = 512


def _vq_body(z_ref, e_ref, et_ref, zq_ref, codes_ref, loss_ref, ppl_ref,
             counts_ref, acc_ref):
    b = pl.program_id(0)
    t = pl.program_id(1)
    nb = pl.num_programs(0)
    nt = pl.num_programs(1)

    z_blk = z_ref[0]
    e = e_ref[...]
    et = et_ref[...]
    n_codes = e.shape[0]

    ze = lax.dot_general(e, z_blk, (((1,), (0,)), ((), ())),
                         preferred_element_type=jnp.float32)
    e2 = jnp.sum(e * e, axis=1, keepdims=True)
    z2 = jnp.sum(z_blk * z_blk, axis=0, keepdims=True)
    dist = (z2 + e2) - 2.0 * ze

    m = jnp.min(dist, axis=0, keepdims=True)
    rows = lax.broadcasted_iota(jnp.int32, dist.shape, 0)
    idx = jnp.min(jnp.where(dist == m, rows, jnp.int32(2**30)), axis=0)
    codes_ref[0, 0, pl.ds(t * KT, KT)] = idx

    oh = (rows == idx[None, :]).astype(jnp.float32)
    zq = lax.dot_general(et, oh, (((1,), (0,)), ((), ())),
                         preferred_element_type=jnp.float32,
                         precision=lax.Precision.HIGHEST)
    zq_ref[0] = zq

    d = zq - z_blk
    psum = jnp.sum(d * d)
    cnt = jnp.sum(oh, axis=1)

    @pl.when((b == 0) & (t == 0))
    def _init():
        counts_ref[...] = cnt
        acc_ref[...] = jnp.full((8, 128), psum, jnp.float32)

    @pl.when((b > 0) | (t > 0))
    def _accum():
        counts_ref[...] = counts_ref[...] + cnt
        acc_ref[...] = acc_ref[...] + psum

    @pl.when((b == nb - 1) & (t == nt - 1))
    def _finalize():
        n_elems = nb * nt * KT
        loss_ref[...] = acc_ref[...] * ((1.0 + BETA) /
                                        (n_elems * z_blk.shape[0]))
        p = counts_ref[...] * (1.0 / n_elems)
        ent = jnp.sum(p * jnp.log(p + 1e-10))
        ppl_ref[...] = jnp.full((8, 128), jnp.exp(-ent), jnp.float32)


def kernel(z_e, codebook):
    B, C, K = z_e.shape
    N = codebook.shape[0]
    et = codebook.T
    grid = (B, K // KT)
    zq, codes3, lossv, pplv = pl.pallas_call(
        _vq_body,
        grid=grid,
        in_specs=[
            pl.BlockSpec((1, C, KT), lambda b, t: (b, 0, t)),
            pl.BlockSpec((N, C), lambda b, t: (0, 0)),
            pl.BlockSpec((C, N), lambda b, t: (0, 0)),
        ],
        out_specs=[
            pl.BlockSpec((1, C, KT), lambda b, t: (b, 0, t)),
            pl.BlockSpec((1, 1, K), lambda b, t: (b, 0, 0)),
            pl.BlockSpec((8, 128), lambda b, t: (0, 0)),
            pl.BlockSpec((8, 128), lambda b, t: (0, 0)),
        ],
        out_shape=[
            jax.ShapeDtypeStruct((B, C, K), jnp.float32),
            jax.ShapeDtypeStruct((B, 1, K), jnp.int32),
            jax.ShapeDtypeStruct((8, 128), jnp.float32),
            jax.ShapeDtypeStruct((8, 128), jnp.float32),
        ],
        scratch_shapes=[
            pltpu.VMEM((N,), jnp.float32),
            pltpu.VMEM((8, 128), jnp.float32),
        ],
        compiler_params=pltpu.CompilerParams(
            dimension_semantics=("arbitrary", "arbitrary")),
    )(z_e, codebook, et)
    return (zq, codes3.reshape(B, K), lossv[0, 0], pplv[0, 0])


# speedup vs baseline: 1.6555x; 1.4983x over previous
import jax
import jax.numpy as jnp
from jax import lax
from jax.experimental import pallas as pl
from jax.experimental.pallas import tpu as pltpu

BETA = 0.25
KT = 512


def _vq_body(z_ref, em2_ref, e_ref, et_ref, zq_ref, codes_ref, loss_ref,
             ppl_ref, counts_ref, acc_ref, e2_ref):
    b = pl.program_id(0)
    t = pl.program_id(1)
    nb = pl.num_programs(0)
    nt = pl.num_programs(1)

    z_blk = z_ref[0]
    em2 = em2_ref[...]
    et = et_ref[...]

    @pl.when((b == 0) & (t == 0))
    def _precompute():
        e = e_ref[...]
        e2_ref[...] = jnp.sum(e * e, axis=1, keepdims=True)

    zem = lax.dot_general(em2, z_blk, (((1,), (0,)), ((), ())),
                          preferred_element_type=jnp.float32)
    e2 = e2_ref[...]
    z2 = jnp.sum(z_blk * z_blk, axis=0, keepdims=True)
    dist = (z2 + e2) + zem

    m = jnp.min(dist, axis=0, keepdims=True)
    rows = lax.broadcasted_iota(jnp.int32, dist.shape, 0)
    idx = jnp.min(jnp.where(dist == m, rows, jnp.int32(2**30)), axis=0)
    codes_ref[0, 0, pl.ds(t * KT, KT)] = idx

    oh = (rows == idx[None, :]).astype(jnp.float32)
    zq = lax.dot_general(et, oh, (((1,), (0,)), ((), ())),
                         preferred_element_type=jnp.float32)
    zq_ref[0] = zq

    d = zq - z_blk
    psum = jnp.sum(d * d)
    cnt = jnp.sum(oh, axis=1)

    @pl.when((b == 0) & (t == 0))
    def _init():
        counts_ref[...] = cnt
        acc_ref[...] = jnp.full((8, 128), psum, jnp.float32)

    @pl.when((b > 0) | (t > 0))
    def _accum():
        counts_ref[...] = counts_ref[...] + cnt
        acc_ref[...] = acc_ref[...] + psum

    @pl.when((b == nb - 1) & (t == nt - 1))
    def _finalize():
        n_elems = nb * nt * KT
        loss_ref[...] = acc_ref[...] * ((1.0 + BETA) /
                                        (n_elems * z_blk.shape[0]))
        p = counts_ref[...] * (1.0 / n_elems)
        ent = jnp.sum(p * jnp.log(p + 1e-10))
        ppl_ref[...] = jnp.full((8, 128), jnp.exp(-ent), jnp.float32)


def kernel(z_e, codebook):
    B, C, K = z_e.shape
    N = codebook.shape[0]
    em2 = -2.0 * codebook
    et = codebook.T
    grid = (B, K // KT)
    zq, codes3, lossv, pplv = pl.pallas_call(
        _vq_body,
        grid=grid,
        in_specs=[
            pl.BlockSpec((1, C, KT), lambda b, t: (b, 0, t)),
            pl.BlockSpec((N, C), lambda b, t: (0, 0)),
            pl.BlockSpec((N, C), lambda b, t: (0, 0)),
            pl.BlockSpec((C, N), lambda b, t: (0, 0)),
        ],
        out_specs=[
            pl.BlockSpec((1, C, KT), lambda b, t: (b, 0, t)),
            pl.BlockSpec((1, 1, K), lambda b, t: (b, 0, 0)),
            pl.BlockSpec((8, 128), lambda b, t: (0, 0)),
            pl.BlockSpec((8, 128), lambda b, t: (0, 0)),
        ],
        out_shape=[
            jax.ShapeDtypeStruct((B, C, K), jnp.float32),
            jax.ShapeDtypeStruct((B, 1, K), jnp.int32),
            jax.ShapeDtypeStruct((8, 128), jnp.float32),
            jax.ShapeDtypeStruct((8, 128), jnp.float32),
        ],
        scratch_shapes=[
            pltpu.VMEM((N,), jnp.float32),
            pltpu.VMEM((8, 128), jnp.float32),
            pltpu.VMEM((N, 1), jnp.float32),
        ],
        compiler_params=pltpu.CompilerParams(
            dimension_semantics=("arbitrary", "arbitrary")),
    )(z_e, em2, codebook, et)
    return (zq, codes3.reshape(B, K), lossv[0, 0], pplv[0, 0])


# speedup vs baseline: 1.8817x; 1.1366x over previous
import jax
import jax.numpy as jnp
from jax import lax
from jax.experimental import pallas as pl
from jax.experimental.pallas import tpu as pltpu

BETA = 0.25
KT = 512


def _vq_body(z_ref, em2_ref, e_ref, et_ref, ones_ref, zq_ref, codes_ref,
             loss_ref, ppl_ref, counts_ref, acc_ref, e2_ref):
    b = pl.program_id(0)
    t = pl.program_id(1)
    nb = pl.num_programs(0)
    nt = pl.num_programs(1)

    z_blk = z_ref[0]
    em2 = em2_ref[...]
    et = et_ref[...]

    @pl.when((b == 0) & (t == 0))
    def _precompute():
        e = e_ref[...]
        e2_ref[...] = jnp.sum(e * e, axis=1, keepdims=True)

    zem = lax.dot_general(em2, z_blk, (((1,), (0,)), ((), ())),
                          preferred_element_type=jnp.float32)
    e2 = e2_ref[...]
    z2 = jnp.sum(z_blk * z_blk, axis=0, keepdims=True)
    dist = (z2 + e2) + zem

    m = jnp.min(dist, axis=0, keepdims=True)
    rows = lax.broadcasted_iota(jnp.int32, dist.shape, 0)
    idx = jnp.min(jnp.where(dist == m, rows, jnp.int32(2**30)), axis=0)
    codes_ref[0, 0, pl.ds(t * KT, KT)] = idx

    oh = (rows == idx[None, :]).astype(jnp.float32)
    zq = lax.dot_general(et, oh, (((1,), (0,)), ((), ())),
                         preferred_element_type=jnp.float32)
    zq_ref[0] = zq

    d = zq - z_blk
    psum = jnp.sum(d * d)
    cnt = lax.dot_general(oh, ones_ref[...], (((1,), (0,)), ((), ())),
                          preferred_element_type=jnp.float32)

    @pl.when((b == 0) & (t == 0))
    def _init():
        counts_ref[...] = cnt
        acc_ref[...] = jnp.full((8, 128), psum, jnp.float32)

    @pl.when((b > 0) | (t > 0))
    def _accum():
        counts_ref[...] = counts_ref[...] + cnt
        acc_ref[...] = acc_ref[...] + psum

    @pl.when((b == nb - 1) & (t == nt - 1))
    def _finalize():
        n_elems = nb * nt * KT
        loss_ref[...] = acc_ref[...] * ((1.0 + BETA) /
                                        (n_elems * z_blk.shape[0]))
        p = counts_ref[:, 0:1] * (1.0 / n_elems)
        ent = jnp.sum(p * jnp.log(p + 1e-10))
        ppl_ref[...] = jnp.full((8, 128), jnp.exp(-ent), jnp.float32)


def kernel(z_e, codebook):
    B, C, K = z_e.shape
    N = codebook.shape[0]
    em2 = -2.0 * codebook
    et = codebook.T
    grid = (B, K // KT)
    zq, codes3, lossv, pplv = pl.pallas_call(
        _vq_body,
        grid=grid,
        in_specs=[
            pl.BlockSpec((1, C, KT), lambda b, t: (b, 0, t)),
            pl.BlockSpec((N, C), lambda b, t: (0, 0)),
            pl.BlockSpec((N, C), lambda b, t: (0, 0)),
            pl.BlockSpec((C, N), lambda b, t: (0, 0)),
            pl.BlockSpec((KT, 128), lambda b, t: (0, 0)),
        ],
        out_specs=[
            pl.BlockSpec((1, C, KT), lambda b, t: (b, 0, t)),
            pl.BlockSpec((1, 1, K), lambda b, t: (b, 0, 0)),
            pl.BlockSpec((8, 128), lambda b, t: (0, 0)),
            pl.BlockSpec((8, 128), lambda b, t: (0, 0)),
        ],
        out_shape=[
            jax.ShapeDtypeStruct((B, C, K), jnp.float32),
            jax.ShapeDtypeStruct((B, 1, K), jnp.int32),
            jax.ShapeDtypeStruct((8, 128), jnp.float32),
            jax.ShapeDtypeStruct((8, 128), jnp.float32),
        ],
        scratch_shapes=[
            pltpu.VMEM((N, 128), jnp.float32),
            pltpu.VMEM((8, 128), jnp.float32),
            pltpu.VMEM((N, 1), jnp.float32),
        ],
        compiler_params=pltpu.CompilerParams(
            dimension_semantics=("arbitrary", "arbitrary")),
    )(z_e, em2, codebook, et, jnp.ones((KT, 128), jnp.float32))
    return (zq, codes3.reshape(B, K), lossv[0, 0], pplv[0, 0])
